# baseline (device time: 65932 ns/iter reference)
import jax
import jax.numpy as jnp
from jax import lax
from jax.experimental import pallas as pl
from jax.experimental.pallas import tpu as pltpu

N_DEV = 16
SQ = 512
HQ = 8
HKV = 2
DH = 128
D = 1024
SCALE = 0.08838834764831843
GQ = HQ // HKV

RS_H = [256, 128, 64, 32]
RS_OFF = [0, 256, 384, 448]


def kernel(x, Wq, Wo, K_ext, V_ext):
    skv = K_ext.shape[1]

    def body(x_ref, wq_ref, wo_ref, k_ref, v_ref, out_ref,
             work_o, work_l, sendb_o, sendb_l, recvb_o, recvb_l, gath,
             rs_ss, rs_rs, rs_ss_l, rs_rs_l, ag_ss, ag_rs):
        my = lax.axis_index("i")
        partners = [my ^ (1 << i) for i in range(4)]

        barrier_sem = pltpu.get_barrier_semaphore()
        for p in partners:
            pl.semaphore_signal(
                barrier_sem, inc=1,
                device_id=(p,), device_id_type=pl.DeviceIdType.MESH,
            )
        pl.semaphore_wait(barrier_sem, 4)

        xb = x_ref[0].astype(jnp.bfloat16)
        wq = wq_ref[...].astype(jnp.bfloat16)
        q = lax.dot_general(
            xb, wq, (((1,), (0,)), ((), ())),
            preferred_element_type=jnp.float32,
        )
        q = (q * SCALE).astype(jnp.bfloat16)

        k2 = k_ref[0].reshape(skv, HKV * DH).astype(jnp.bfloat16)
        v2 = v_ref[0].reshape(skv, HKV * DH).astype(jnp.bfloat16)

        o_parts = []
        l_parts = []
        for h in range(HQ):
            g = h // GQ
            qh = q[:, h * DH:(h + 1) * DH]
            kg = k2[:, g * DH:(g + 1) * DH]
            vg = v2[:, g * DH:(g + 1) * DH]
            s = lax.dot_general(
                qh, kg, (((1,), (1,)), ((), ())),
                preferred_element_type=jnp.float32,
            )
            p = jnp.exp(s)
            l_parts.append(jnp.sum(p, axis=1, keepdims=True))
            o_parts.append(lax.dot_general(
                p.astype(jnp.bfloat16), vg, (((1,), (0,)), ((), ())),
                preferred_element_type=jnp.float32,
            ))
        work_o[:, :] = jnp.concatenate(o_parts, axis=1)
        work_l[:, :] = jnp.concatenate(l_parts, axis=1)

        base = my * 0
        for i in range(4):
            h = RS_H[i]
            off = RS_OFF[i]
            bit = (my >> i) & 1
            send_start = pl.multiple_of(base + (1 - bit) * h, 32)
            keep_start = pl.multiple_of(base + bit * h, 32)
            sendb_o[pl.ds(off, h), :] = (
                work_o[pl.ds(send_start, h), :].astype(jnp.bfloat16)
            )
            sendb_l[pl.ds(off, h), :] = work_l[pl.ds(send_start, h), :]
            rdma_o = pltpu.make_async_remote_copy(
                src_ref=sendb_o.at[pl.ds(off, h)],
                dst_ref=recvb_o.at[pl.ds(off, h)],
                send_sem=rs_ss.at[i],
                recv_sem=rs_rs.at[i],
                device_id=(partners[i],),
                device_id_type=pl.DeviceIdType.MESH,
            )
            rdma_l = pltpu.make_async_remote_copy(
                src_ref=sendb_l.at[pl.ds(off, h)],
                dst_ref=recvb_l.at[pl.ds(off, h)],
                send_sem=rs_ss_l.at[i],
                recv_sem=rs_rs_l.at[i],
                device_id=(partners[i],),
                device_id_type=pl.DeviceIdType.MESH,
            )
            rdma_o.start()
            rdma_l.start()
            rdma_o.wait()
            rdma_l.wait()
            work_o[pl.ds(keep_start, h), :] = (
                work_o[pl.ds(keep_start, h), :]
                + recvb_o[pl.ds(off, h), :].astype(jnp.float32)
            )
            work_l[pl.ds(keep_start, h), :] = (
                work_l[pl.ds(keep_start, h), :] + recvb_l[pl.ds(off, h), :]
            )
            base = keep_start

        o32 = work_o[pl.ds(base, 32), :]
        l32 = work_l[pl.ds(base, 32), :]
        recip = 1.0 / l32
        on = jnp.concatenate(
            [o32[:, h * DH:(h + 1) * DH] * recip[:, h:h + 1]
             for h in range(HQ)],
            axis=1,
        ).astype(jnp.bfloat16)
        wo = wo_ref[...].astype(jnp.bfloat16)
        y32 = lax.dot_general(
            on, wo, (((1,), (0,)), ((), ())),
            preferred_element_type=jnp.float32,
        )
        gath[pl.ds(base, 32), :] = y32.astype(jnp.bfloat16)

        gbase = base
        for j in range(4):
            size = 32 << j
            partner = my ^ (1 << (3 - j))
            rdma = pltpu.make_async_remote_copy(
                src_ref=gath.at[pl.ds(gbase, size)],
                dst_ref=gath.at[pl.ds(gbase, size)],
                send_sem=ag_ss.at[j],
                recv_sem=ag_rs.at[j],
                device_id=(partner,),
                device_id_type=pl.DeviceIdType.MESH,
            )
            rdma.start()
            rdma.wait()
            sib = gbase ^ size
            gbase = pl.multiple_of(jnp.minimum(gbase, sib), 32)

        out_ref[0] = gath[:, :].astype(jnp.float32)

    return pl.pallas_call(
        body,
        out_shape=jax.ShapeDtypeStruct((1, SQ, D), jnp.float32),
        in_specs=[pl.BlockSpec(memory_space=pltpu.VMEM)] * 5,
        out_specs=pl.BlockSpec(memory_space=pltpu.VMEM),
        scratch_shapes=[
            pltpu.VMEM((SQ, D), jnp.float32),
            pltpu.VMEM((SQ, HQ), jnp.float32),
            pltpu.VMEM((480, D), jnp.bfloat16),
            pltpu.VMEM((480, HQ), jnp.float32),
            pltpu.VMEM((480, D), jnp.bfloat16),
            pltpu.VMEM((480, HQ), jnp.float32),
            pltpu.VMEM((SQ, D), jnp.bfloat16),
            pltpu.SemaphoreType.DMA((4,)),
            pltpu.SemaphoreType.DMA((4,)),
            pltpu.SemaphoreType.DMA((4,)),
            pltpu.SemaphoreType.DMA((4,)),
            pltpu.SemaphoreType.DMA((4,)),
            pltpu.SemaphoreType.DMA((4,)),
        ],
        compiler_params=pltpu.CompilerParams(collective_id=0),
    )(x, Wq, Wo, K_ext, V_ext)


# device time: 52760 ns/iter; 1.2497x vs baseline; 1.2497x over previous
import jax
import jax.numpy as jnp
from jax import lax
from jax.experimental import pallas as pl
from jax.experimental.pallas import tpu as pltpu

N_DEV = 16
BLK = 32
SQ = 512
HQ = 8
HKV = 2
DH = 128
D = 1024
SCALE = 0.08838834764831843
GQ = HQ // HKV


def kernel(x, Wq, Wo, K_ext, V_ext):
    skv = K_ext.shape[1]

    def body(x_ref, wq_ref, wo_ref, k_ref, v_ref, out_ref,
             sendb_o, sendb_l, recvb_o, recvb_l, gath,
             rso_ss, rso_rs, rsl_ss, rsl_rs, ag_ss, ag_rs):
        my = lax.axis_index("i")
        mybase = pl.multiple_of(my * BLK, BLK)

        barrier_sem = pltpu.get_barrier_semaphore()
        for j in range(N_DEV):
            @pl.when(my != j)
            def _():
                pl.semaphore_signal(
                    barrier_sem, inc=1,
                    device_id=(j,), device_id_type=pl.DeviceIdType.MESH,
                )
        pl.semaphore_wait(barrier_sem, N_DEV - 1)

        xb = x_ref[0].astype(jnp.bfloat16)
        wq = wq_ref[...].astype(jnp.bfloat16)
        q = lax.dot_general(
            xb, wq, (((1,), (0,)), ((), ())),
            preferred_element_type=jnp.float32,
        )
        q = (q * SCALE).astype(jnp.bfloat16)

        k2 = k_ref[0].reshape(skv, HKV * DH).astype(jnp.bfloat16)
        v2 = v_ref[0].reshape(skv, HKV * DH).astype(jnp.bfloat16)

        o_parts = []
        l_parts = []
        for h in range(HQ):
            g = h // GQ
            qh = q[:, h * DH:(h + 1) * DH]
            kg = k2[:, g * DH:(g + 1) * DH]
            vg = v2[:, g * DH:(g + 1) * DH]
            s = lax.dot_general(
                qh, kg, (((1,), (1,)), ((), ())),
                preferred_element_type=jnp.float32,
            )
            p = jnp.exp(s)
            l_parts.append(jnp.sum(p, axis=1, keepdims=True))
            o_parts.append(lax.dot_general(
                p.astype(jnp.bfloat16), vg, (((1,), (0,)), ((), ())),
                preferred_element_type=jnp.float32,
            ))
        o_loc = jnp.concatenate(o_parts, axis=1)
        l_loc = jnp.concatenate(l_parts, axis=1)
        sendb_o[:, :] = o_loc.astype(jnp.bfloat16)
        sendb_l[:, :] = l_loc

        def rs_desc(j):
            return (
                pltpu.make_async_remote_copy(
                    src_ref=sendb_o.at[pl.ds(j * BLK, BLK)],
                    dst_ref=recvb_o.at[pl.ds(mybase, BLK)],
                    send_sem=rso_ss.at[j],
                    recv_sem=rso_rs.at[my],
                    device_id=(j,),
                    device_id_type=pl.DeviceIdType.MESH,
                ),
                pltpu.make_async_remote_copy(
                    src_ref=sendb_l.at[pl.ds(j * BLK, BLK)],
                    dst_ref=recvb_l.at[pl.ds(mybase, BLK)],
                    send_sem=rsl_ss.at[j],
                    recv_sem=rsl_rs.at[my],
                    device_id=(j,),
                    device_id_type=pl.DeviceIdType.MESH,
                ),
            )

        for j in range(N_DEV):
            @pl.when(my != j)
            def _():
                d_o, d_l = rs_desc(j)
                d_o.start()
                d_l.start()

        recvb_o[pl.ds(mybase, BLK), :] = sendb_o[pl.ds(mybase, BLK), :]
        recvb_l[pl.ds(mybase, BLK), :] = sendb_l[pl.ds(mybase, BLK), :]

        for j in range(N_DEV):
            @pl.when(my != j)
            def _():
                recv_o = pltpu.make_async_remote_copy(
                    src_ref=sendb_o.at[pl.ds(0, BLK)],
                    dst_ref=recvb_o.at[pl.ds(j * BLK, BLK)],
                    send_sem=rso_ss.at[j],
                    recv_sem=rso_rs.at[j],
                    device_id=(j,),
                    device_id_type=pl.DeviceIdType.MESH,
                )
                recv_l = pltpu.make_async_remote_copy(
                    src_ref=sendb_l.at[pl.ds(0, BLK)],
                    dst_ref=recvb_l.at[pl.ds(j * BLK, BLK)],
                    send_sem=rsl_ss.at[j],
                    recv_sem=rsl_rs.at[j],
                    device_id=(j,),
                    device_id_type=pl.DeviceIdType.MESH,
                )
                recv_o.wait_recv()
                recv_l.wait_recv()

        o32 = recvb_o[0:BLK, :].astype(jnp.float32)
        l32 = recvb_l[0:BLK, :]
        for j in range(1, N_DEV):
            o32 = o32 + recvb_o[j * BLK:(j + 1) * BLK, :].astype(jnp.float32)
            l32 = l32 + recvb_l[j * BLK:(j + 1) * BLK, :]

        recip = 1.0 / l32
        on = jnp.concatenate(
            [o32[:, h * DH:(h + 1) * DH] * recip[:, h:h + 1]
             for h in range(HQ)],
            axis=1,
        ).astype(jnp.bfloat16)
        wo = wo_ref[...].astype(jnp.bfloat16)
        y32 = lax.dot_general(
            on, wo, (((1,), (0,)), ((), ())),
            preferred_element_type=jnp.float32,
        )
        gath[pl.ds(mybase, BLK), :] = y32.astype(jnp.bfloat16)

        def ag_desc(j):
            return pltpu.make_async_remote_copy(
                src_ref=gath.at[pl.ds(mybase, BLK)],
                dst_ref=gath.at[pl.ds(mybase, BLK)],
                send_sem=ag_ss.at[j],
                recv_sem=ag_rs.at[my],
                device_id=(j,),
                device_id_type=pl.DeviceIdType.MESH,
            )

        for j in range(N_DEV):
            @pl.when(my != j)
            def _():
                ag_desc(j).start()

        for j in range(N_DEV):
            @pl.when(my != j)
            def _():
                recv = pltpu.make_async_remote_copy(
                    src_ref=gath.at[pl.ds(0, BLK)],
                    dst_ref=gath.at[pl.ds(j * BLK, BLK)],
                    send_sem=ag_ss.at[j],
                    recv_sem=ag_rs.at[j],
                    device_id=(j,),
                    device_id_type=pl.DeviceIdType.MESH,
                )
                recv.wait_recv()

        out_ref[0] = gath[:, :].astype(jnp.float32)

        for j in range(N_DEV):
            @pl.when(my != j)
            def _():
                d_o, d_l = rs_desc(j)
                d_o.wait_send()
                d_l.wait_send()
                ag_desc(j).wait_send()

    return pl.pallas_call(
        body,
        out_shape=jax.ShapeDtypeStruct((1, SQ, D), jnp.float32),
        in_specs=[pl.BlockSpec(memory_space=pltpu.VMEM)] * 5,
        out_specs=pl.BlockSpec(memory_space=pltpu.VMEM),
        scratch_shapes=[
            pltpu.VMEM((SQ, D), jnp.bfloat16),
            pltpu.VMEM((SQ, HQ), jnp.float32),
            pltpu.VMEM((SQ, D), jnp.bfloat16),
            pltpu.VMEM((SQ, HQ), jnp.float32),
            pltpu.VMEM((SQ, D), jnp.bfloat16),
            pltpu.SemaphoreType.DMA((N_DEV,)),
            pltpu.SemaphoreType.DMA((N_DEV,)),
            pltpu.SemaphoreType.DMA((N_DEV,)),
            pltpu.SemaphoreType.DMA((N_DEV,)),
            pltpu.SemaphoreType.DMA((N_DEV,)),
            pltpu.SemaphoreType.DMA((N_DEV,)),
        ],
        compiler_params=pltpu.CompilerParams(collective_id=0),
    )(x, Wq, Wo, K_ext, V_ext)
